# baseline (speedup 1.0000x reference)
import functools

import jax
import jax.numpy as jnp
from jax import lax
from jax.experimental import pallas as pl
from jax.experimental.pallas import tpu as pltpu
from jax.experimental.pallas import tpu_sc as plsc

BATCH = 16384
HIST = 50
EMBED_DIM = 64

NC = 2
NS = 16
NW = NC * NS
BPW = BATCH // NW
C = 256
NHALF = BPW // C
NCHUNK = HIST * NHALF

_mesh = plsc.VectorSubcoreMesh(
    core_axis_name="c", subcore_axis_name="s", num_cores=NC, num_subcores=NS
)


@functools.partial(
    pl.kernel,
    out_type=jax.ShapeDtypeStruct((HIST, EMBED_DIM, BATCH), jnp.float32),
    mesh=_mesh,
    compiler_params=pltpu.CompilerParams(
        use_tc_tiling_on_sc=True, needs_layout_passes=False
    ),
    scratch_types=[
        pltpu.VMEM((HIST, BPW), jnp.int32),
        pltpu.VMEM((C,), jnp.int32),
        pltpu.VMEM((C,), jnp.int32),
        pltpu.VMEM((C, 128), jnp.float32),
        pltpu.VMEM((C, 128), jnp.float32),
        pltpu.VMEM((EMBED_DIM, C), jnp.float32),
        pltpu.VMEM((EMBED_DIM, C), jnp.float32),
        pltpu.SemaphoreType.DMA,
        pltpu.SemaphoreType.DMA,
        pltpu.SemaphoreType.DMA,
        pltpu.SemaphoreType.DMA,
    ],
)
def _embed_gather(xt_hbm, t2_hbm, out_hbm, idx_v, pidx0, pidx1, rows0, rows1,
                  to0, to1, gsem0, gsem1, osem0, osem1):
    wid = lax.axis_index("s") * NC + lax.axis_index("c")
    b0 = wid * BPW

    pidx = (pidx0, pidx1)
    rows = (rows0, rows1)
    to = (to0, to1)
    gsem = (gsem0, gsem1)
    osem = (osem0, osem1)

    pltpu.sync_copy(xt_hbm.at[:, pl.ds(b0, BPW)], idx_v)

    iota = lax.iota(jnp.int32, 16)

    def pidx_compute(c, buf):
        h = c // NHALF
        off = (c % NHALF) * C
        for k in range(C // 16):
            v = idx_v[h, pl.ds(off + k * 16, 16)]
            pidx[buf][pl.ds(k * 16, 16)] = lax.shift_right_logical(v, 1)

    def gather_start(buf):
        pltpu.make_async_copy(t2_hbm.at[pidx[buf]], rows[buf], gsem[buf]).start()

    def gather_wait(buf):
        pltpu.make_async_copy(t2_hbm.at[pidx[buf]], rows[buf], gsem[buf]).wait()

    def out_start(c, buf):
        h = c // NHALF
        off = (c % NHALF) * C
        pltpu.make_async_copy(
            to[buf], out_hbm.at[h, :, pl.ds(b0 + off, C)], osem[buf]
        ).start()

    def out_wait(c, buf):
        h = c // NHALF
        off = (c % NHALF) * C
        pltpu.make_async_copy(
            to[buf], out_hbm.at[h, :, pl.ds(b0 + off, C)], osem[buf]
        ).wait()

    def transpose_chunk(c, buf):
        h = c // NHALF
        off = (c % NHALF) * C

        @pl.loop(0, C // 16)
        def _kb(kb):
            v = idx_v[h, pl.ds(off + kb * 16, 16)]
            par64 = (v & 1) * 64
            row_ids = kb * 16 + iota
            for e in range(EMBED_DIM):
                col_ids = par64 + e
                vals = plsc.load_gather(rows[buf], [row_ids, col_ids])
                to[buf][e, pl.ds(kb * 16, 16)] = vals

    pidx_compute(0, 0)
    gather_start(0)

    @pl.loop(0, NCHUNK, step=2)
    def _main(c):
        for half in range(2):
            cc = c + half
            buf = half
            nbuf = 1 - half

            @pl.when(cc + 1 < NCHUNK)
            def _():
                pidx_compute(cc + 1, nbuf)
                gather_start(nbuf)

            gather_wait(buf)

            @pl.when(cc >= 2)
            def _():
                out_wait(cc - 2, buf)

            transpose_chunk(cc, buf)
            out_start(cc, buf)

    out_wait(NCHUNK - 2, 0)
    out_wait(NCHUNK - 1, 1)


def kernel(x, table):
    xt = x.T.astype(jnp.int32)
    t2 = table.reshape(500000, 128)
    out_t = _embed_gather(xt, t2)
    return out_t.transpose(2, 0, 1)


# speedup vs baseline: 1.4712x; 1.4712x over previous
import functools

import jax
import jax.numpy as jnp
from jax import lax
from jax.experimental import pallas as pl
from jax.experimental.pallas import tpu as pltpu
from jax.experimental.pallas import tpu_sc as plsc

BATCH = 16384
HIST = 50
EMBED_DIM = 64
NTOT = BATCH * HIST

NC = 2
NS = 16
NW = NC * NS
BPW = NTOT // NW

C = 256
NCH = BPW // C
NBUF = 5
K = 3

_mesh = plsc.VectorSubcoreMesh(
    core_axis_name="c", subcore_axis_name="s", num_cores=NC, num_subcores=NS
)


@functools.partial(
    pl.kernel,
    out_type=jax.ShapeDtypeStruct((NTOT, EMBED_DIM), jnp.float32),
    mesh=_mesh,
    compiler_params=pltpu.CompilerParams(use_tc_tiling_on_sc=False),
    scratch_types=[
        pltpu.VMEM((NCH, C), jnp.int32),
        pltpu.VMEM((NBUF, C, EMBED_DIM), jnp.float32),
        pltpu.SemaphoreType.DMA((NBUF,)),
        pltpu.SemaphoreType.DMA((NBUF,)),
    ],
)
def _embed_gather(x_hbm, table_hbm, out_hbm, idx_v, rows_v, gsem, osem):
    wid = lax.axis_index("s") * NC + lax.axis_index("c")
    chunk0 = wid * NCH

    pltpu.sync_copy(x_hbm.at[pl.ds(chunk0, NCH)], idx_v)

    def gather_start(ch, b):
        pltpu.make_async_copy(
            table_hbm.at[idx_v.at[ch]], rows_v.at[b], gsem.at[b]
        ).start()

    def gather_wait(ch, b):
        pltpu.make_async_copy(
            table_hbm.at[idx_v.at[ch]], rows_v.at[b], gsem.at[b]
        ).wait()

    def out_start(ch, b):
        pltpu.make_async_copy(
            rows_v.at[b], out_hbm.at[pl.ds((chunk0 + ch) * C, C)], osem.at[b]
        ).start()

    def out_wait(ch, b):
        pltpu.make_async_copy(
            rows_v.at[b], out_hbm.at[pl.ds((chunk0 + ch) * C, C)], osem.at[b]
        ).wait()

    for i in range(NBUF):
        gather_start(i, i)
        if i >= K:
            gather_wait(i - K, i - K)
            out_start(i - K, i - K)

    @pl.loop(NBUF, NCH, step=NBUF)
    def _main(g):
        for j in range(NBUF):
            i = g + j
            out_wait(i - NBUF, j)
            gather_start(i, j)
            gather_wait(i - K, (j - K) % NBUF)
            out_start(i - K, (j - K) % NBUF)

    for i in range(NCH, NCH + K):
        gather_wait(i - K, (i - K) % NBUF)
        out_start(i - K, (i - K) % NBUF)
    for ch in range(NCH - NBUF, NCH):
        out_wait(ch, ch % NBUF)


def kernel(x, table):
    x2d = x.reshape(NW * NCH, C).astype(jnp.int32)
    out = _embed_gather(x2d, table)
    return out.reshape(BATCH, HIST, EMBED_DIM)
